# baseline (speedup 1.0000x reference)
import functools

import jax
import jax.numpy as jnp
from jax import lax
from jax.experimental import pallas as pl
from jax.experimental.pallas import tpu as pltpu
from jax.experimental.pallas import tpu_sc as plsc

NODES = 10000
EDGES = 320000
D = 16

NC = 2
NS = 16
NW = NC * NS
CH = 128
NCH = 80
EPAD = NW * NCH * CH
RPT = 640
NP = NS * RPT
NBUF = 16
WROWS = NP * D // 128
NPACK = 128 // D

_SC_PARAMS = pltpu.CompilerParams(use_tc_tiling_on_sc=False)

_SC_SCRATCH = [
    pltpu.VMEM((NCH, CH), jnp.int32),
    pltpu.VMEM((NCH, CH), jnp.int32),
    pltpu.VMEM((NBUF, CH, D), jnp.float32),
    pltpu.VMEM((CH, D), jnp.float32),
    pltpu.VMEM_SHARED((NP, D), jnp.float32),
    pltpu.VMEM_SHARED((NP, D), jnp.float32),
    tuple(pltpu.SemaphoreType.DMA for _ in range(NBUF)),
]


def _zero_accumulator(zc_hbm, zero_v, agg_sh, sid):
    pltpu.sync_copy(zc_hbm, zero_v)

    @pl.loop(0, RPT, step=CH)
    def _(r):
        pltpu.sync_copy(zero_v, agg_sh.at[pl.ds(sid * RPT + r, CH)])


def _seg_sum_loop(table_sh, src_v, dst_v, rows_v, agg_sh, sems):
    for b in range(NBUF):
        pltpu.async_copy(table_sh.at[src_v.at[b]], rows_v.at[b], sems[b])

    @pl.loop(0, NCH, step=NBUF)
    def _(j):
        for b in range(NBUF):
            c = j + b
            pltpu.make_async_copy(
                table_sh.at[src_v.at[c]], rows_v.at[b], sems[b]).wait()
            pltpu.sync_copy(rows_v.at[b], agg_sh.at[dst_v.at[c]], add=True)

            @pl.when(c + NBUF < NCH)
            def _():
                pltpu.async_copy(table_sh.at[src_v.at[c + NBUF]],
                                 rows_v.at[b], sems[b])


def _seg_sum_sc(table, ei, zeros_chunk):
    mesh = plsc.VectorSubcoreMesh(core_axis_name="c", subcore_axis_name="s")

    @functools.partial(
        pl.kernel,
        out_type=jax.ShapeDtypeStruct((NC, NP, D), jnp.float32),
        mesh=mesh,
        compiler_params=_SC_PARAMS,
        scratch_types=_SC_SCRATCH,
    )
    def k(table_hbm, ei_hbm, zc_hbm, out_hbm,
          src_v, dst_v, rows_v, zero_v, agg_sh, table_sh, sems):
        cid = lax.axis_index("c")
        sid = lax.axis_index("s")
        wid = cid * NS + sid

        pltpu.sync_copy(ei_hbm.at[0, wid], src_v)
        pltpu.sync_copy(ei_hbm.at[1, wid], dst_v)
        pltpu.sync_copy(table_hbm.at[pl.ds(sid * RPT, RPT)],
                        table_sh.at[pl.ds(sid * RPT, RPT)])
        _zero_accumulator(zc_hbm, zero_v, agg_sh, sid)
        plsc.subcore_barrier()
        _seg_sum_loop(table_sh, src_v, dst_v, rows_v, agg_sh, sems)
        plsc.subcore_barrier()
        pltpu.sync_copy(agg_sh.at[pl.ds(sid * RPT, RPT)],
                        out_hbm.at[cid, pl.ds(sid * RPT, RPT)])

    return k(table, ei, zeros_chunk)


def _relu_seg_sum_sc(parts1, r1, b1, ei, zeros_chunk):
    mesh = plsc.VectorSubcoreMesh(core_axis_name="c", subcore_axis_name="s")

    @functools.partial(
        pl.kernel,
        out_type=(jax.ShapeDtypeStruct((NC, NP, D), jnp.float32),
                  jax.ShapeDtypeStruct((NP, D), jnp.float32)),
        mesh=mesh,
        compiler_params=_SC_PARAMS,
        scratch_types=_SC_SCRATCH + [
            pltpu.VMEM((RPT, D), jnp.float32),
            pltpu.VMEM((RPT, D), jnp.float32),
            pltpu.VMEM((RPT, D), jnp.float32),
            pltpu.VMEM((D,), jnp.float32),
        ],
    )
    def k(parts_hbm, r1_hbm, b1_hbm, ei_hbm, zc_hbm, out_hbm, h_hbm,
          src_v, dst_v, rows_v, zero_v, agg_sh, table_sh, sems,
          pa_v, pb_v, ph_v, b1_v):
        cid = lax.axis_index("c")
        sid = lax.axis_index("s")
        wid = cid * NS + sid

        pltpu.sync_copy(ei_hbm.at[0, wid], src_v)
        pltpu.sync_copy(ei_hbm.at[1, wid], dst_v)

        pltpu.sync_copy(parts_hbm.at[0, pl.ds(sid * RPT, RPT)], pa_v)
        pltpu.sync_copy(parts_hbm.at[1, pl.ds(sid * RPT, RPT)], pb_v)
        pltpu.sync_copy(r1_hbm.at[pl.ds(sid * RPT, RPT)], ph_v)
        pltpu.sync_copy(b1_hbm, b1_v)
        bias = b1_v[...]

        @pl.loop(0, RPT, step=4)
        def _(i0):
            for di in range(4):
                i = i0 + di
                ph_v[i, :] = jnp.maximum(
                    pa_v[i, :] + pb_v[i, :] + ph_v[i, :] + bias, 0.0)

        pltpu.sync_copy(ph_v, table_sh.at[pl.ds(sid * RPT, RPT)])

        @pl.when(cid == 0)
        def _():
            pltpu.sync_copy(ph_v, h_hbm.at[pl.ds(sid * RPT, RPT)])

        _zero_accumulator(zc_hbm, zero_v, agg_sh, sid)
        plsc.subcore_barrier()
        _seg_sum_loop(table_sh, src_v, dst_v, rows_v, agg_sh, sems)
        plsc.subcore_barrier()
        pltpu.sync_copy(agg_sh.at[pl.ds(sid * RPT, RPT)],
                        out_hbm.at[cid, pl.ds(sid * RPT, RPT)])

    return k(parts1, r1, b1, ei, zeros_chunk)


def _tc_pre(x_w, w_rel_blk, w_root_blk):
    def body(x_ref, a_ref, b_ref, p_ref, r_ref):
        xv = x_ref[...]
        p_ref[...] = jnp.dot(xv, a_ref[...], preferred_element_type=jnp.float32)
        r_ref[...] = jnp.dot(xv, b_ref[...], preferred_element_type=jnp.float32)

    return pl.pallas_call(
        body,
        out_shape=(jax.ShapeDtypeStruct((WROWS, 128), jnp.float32),
                   jax.ShapeDtypeStruct((WROWS, 128), jnp.float32)),
    )(x_w, w_rel_blk, w_root_blk)


def _tc_post(parts_w, h_w, w2_rel_blk, b2_tile, w2_root_blk):
    def body(parts_ref, h_ref, wr_ref, b_ref, wo_ref, o_ref):
        agg_w = parts_ref[0] + parts_ref[1]
        o_ref[...] = (
            jnp.dot(agg_w, wr_ref[...], preferred_element_type=jnp.float32)
            + jnp.dot(h_ref[...], wo_ref[...],
                      preferred_element_type=jnp.float32)
            + b_ref[...])

    return pl.pallas_call(
        body,
        out_shape=jax.ShapeDtypeStruct((WROWS, 2 * NPACK), jnp.float32),
    )(parts_w, h_w, w2_rel_blk, b2_tile, w2_root_blk)


def kernel(x, edge_index, W1_rel, b1, W1_root, W2_rel, b2, W2_root):
    npad = EPAD - EDGES
    ei_pad = jnp.concatenate(
        [jnp.zeros((1, npad), jnp.int32),
         jnp.full((1, npad), NODES, jnp.int32)])
    ei = jnp.concatenate(
        [edge_index.astype(jnp.int32), ei_pad], axis=1).reshape(
            2, NW, NCH, CH)
    zeros_chunk = jnp.zeros((CH, D), jnp.float32)
    eye8 = jnp.eye(NPACK, dtype=jnp.float32)
    w1_rel_blk = jnp.kron(eye8, W1_rel)
    w1_root_blk = jnp.kron(eye8, W1_root)
    w2_rel_blk = jnp.kron(eye8, W2_rel)
    w2_root_blk = jnp.kron(eye8, W2_root)
    b2_tile = jnp.tile(b2, NPACK)
    x_w = jnp.concatenate(
        [x, jnp.zeros((NP - NODES, x.shape[1]), jnp.float32)]).reshape(
            WROWS, NPACK * x.shape[1])

    p1w, r1w = _tc_pre(x_w, w1_rel_blk, w1_root_blk)
    parts1 = _seg_sum_sc(p1w.reshape(NP, D), ei, zeros_chunk)
    parts2, h = _relu_seg_sum_sc(
        parts1, r1w.reshape(NP, D), b1, ei, zeros_chunk)
    out_w = _tc_post(parts2.reshape(NC, WROWS, 128), h.reshape(WROWS, 128),
                     w2_rel_blk, b2_tile, w2_root_blk)
    return out_w.reshape(NP * 2)[:NODES * 2].reshape(NODES, 2)


# speedup vs baseline: 1.0699x; 1.0699x over previous
import functools

import jax
import jax.numpy as jnp
from jax import lax
from jax.experimental import pallas as pl
from jax.experimental.pallas import tpu as pltpu
from jax.experimental.pallas import tpu_sc as plsc

NODES = 10000
EDGES = 320000
D = 16

NC = 2
NS = 16
NW = NC * NS
CH = 128
NCH = 80
EPAD = NW * NCH * CH
RPT = 640
NP = NS * RPT
NBUF = 8
WROWS = NP * D // 128
NPACK = 128 // D

_SC_PARAMS = pltpu.CompilerParams(use_tc_tiling_on_sc=False)

_SC_SCRATCH = [
    pltpu.VMEM((NCH, CH), jnp.int32),
    pltpu.VMEM((NCH, CH), jnp.int32),
    pltpu.VMEM((NBUF, CH, D), jnp.float32),
    pltpu.VMEM((CH, D), jnp.float32),
    pltpu.VMEM_SHARED((NP, D), jnp.float32),
    pltpu.VMEM_SHARED((NP, D), jnp.float32),
    tuple(pltpu.SemaphoreType.DMA for _ in range(NBUF)),
]


NZC = RPT // CH


def _issue_zero(zero_v, agg_sh, sid, sems, s0):
    for i in range(NZC):
        pltpu.async_copy(
            zero_v, agg_sh.at[pl.ds(sid * RPT + i * CH, CH)], sems[s0 + i])


def _wait_zero(zero_v, agg_sh, sid, sems, s0):
    for i in range(NZC):
        pltpu.make_async_copy(
            zero_v, agg_sh.at[pl.ds(sid * RPT + i * CH, CH)],
            sems[s0 + i]).wait()


def _seg_sum_loop(table_sh, src_v, dst_v, rows_v, agg_sh, sems):
    for b in range(NBUF):
        pltpu.async_copy(table_sh.at[src_v.at[b]], rows_v.at[b], sems[b])

    @pl.loop(0, NCH, step=NBUF)
    def _(j):
        for b in range(NBUF):
            c = j + b
            pltpu.make_async_copy(
                table_sh.at[src_v.at[c]], rows_v.at[b], sems[b]).wait()
            pltpu.sync_copy(rows_v.at[b], agg_sh.at[dst_v.at[c]], add=True)

            @pl.when(c + NBUF < NCH)
            def _():
                pltpu.async_copy(table_sh.at[src_v.at[c + NBUF]],
                                 rows_v.at[b], sems[b])


def _seg_sum_sc(table, ei, zeros_chunk):
    mesh = plsc.VectorSubcoreMesh(core_axis_name="c", subcore_axis_name="s")

    @functools.partial(
        pl.kernel,
        out_type=jax.ShapeDtypeStruct((NC, NP, D), jnp.float32),
        mesh=mesh,
        compiler_params=_SC_PARAMS,
        scratch_types=_SC_SCRATCH,
    )
    def k(table_hbm, ei_hbm, zc_hbm, out_hbm,
          src_v, dst_v, rows_v, zero_v, agg_sh, table_sh, sems):
        cid = lax.axis_index("c")
        sid = lax.axis_index("s")
        wid = cid * NS + sid

        tsl = pl.ds(sid * RPT, RPT)
        pltpu.async_copy(ei_hbm.at[0, wid], src_v, sems[0])
        pltpu.async_copy(ei_hbm.at[1, wid], dst_v, sems[1])
        pltpu.async_copy(table_hbm.at[tsl], table_sh.at[tsl], sems[2])
        pltpu.async_copy(zc_hbm, zero_v, sems[3])
        pltpu.make_async_copy(zc_hbm, zero_v, sems[3]).wait()
        _issue_zero(zero_v, agg_sh, sid, sems, 3)
        pltpu.make_async_copy(ei_hbm.at[0, wid], src_v, sems[0]).wait()
        pltpu.make_async_copy(ei_hbm.at[1, wid], dst_v, sems[1]).wait()
        pltpu.make_async_copy(
            table_hbm.at[tsl], table_sh.at[tsl], sems[2]).wait()
        _wait_zero(zero_v, agg_sh, sid, sems, 3)
        plsc.subcore_barrier()
        _seg_sum_loop(table_sh, src_v, dst_v, rows_v, agg_sh, sems)
        plsc.subcore_barrier()
        pltpu.sync_copy(agg_sh.at[pl.ds(sid * RPT, RPT)],
                        out_hbm.at[cid, pl.ds(sid * RPT, RPT)])

    return k(table, ei, zeros_chunk)


def _relu_seg_sum_sc(parts1, r1, b1, ei, zeros_chunk):
    mesh = plsc.VectorSubcoreMesh(core_axis_name="c", subcore_axis_name="s")

    @functools.partial(
        pl.kernel,
        out_type=(jax.ShapeDtypeStruct((NC, NP, D), jnp.float32),
                  jax.ShapeDtypeStruct((NP, D), jnp.float32)),
        mesh=mesh,
        compiler_params=_SC_PARAMS,
        scratch_types=_SC_SCRATCH + [
            pltpu.VMEM((RPT, D), jnp.float32),
            pltpu.VMEM((RPT, D), jnp.float32),
            pltpu.VMEM((RPT, D), jnp.float32),
            pltpu.VMEM((D,), jnp.float32),
        ],
    )
    def k(parts_hbm, r1_hbm, b1_hbm, ei_hbm, zc_hbm, out_hbm, h_hbm,
          src_v, dst_v, rows_v, zero_v, agg_sh, table_sh, sems,
          pa_v, pb_v, ph_v, b1_v):
        cid = lax.axis_index("c")
        sid = lax.axis_index("s")
        wid = cid * NS + sid

        tsl = pl.ds(sid * RPT, RPT)
        pltpu.async_copy(ei_hbm.at[0, wid], src_v, sems[0])
        pltpu.async_copy(ei_hbm.at[1, wid], dst_v, sems[1])
        pltpu.async_copy(parts_hbm.at[0, tsl], pa_v, sems[2])
        pltpu.async_copy(parts_hbm.at[1, tsl], pb_v, sems[3])
        pltpu.async_copy(r1_hbm.at[tsl], ph_v, sems[4])
        pltpu.async_copy(b1_hbm, b1_v, sems[5])
        pltpu.async_copy(zc_hbm, zero_v, sems[6])
        pltpu.make_async_copy(zc_hbm, zero_v, sems[6]).wait()
        pltpu.make_async_copy(parts_hbm.at[0, tsl], pa_v, sems[2]).wait()
        pltpu.make_async_copy(parts_hbm.at[1, tsl], pb_v, sems[3]).wait()
        pltpu.make_async_copy(r1_hbm.at[tsl], ph_v, sems[4]).wait()
        pltpu.make_async_copy(b1_hbm, b1_v, sems[5]).wait()
        _issue_zero(zero_v, agg_sh, sid, sems, 2)
        bias = b1_v[...]

        @pl.loop(0, RPT, step=4)
        def _(i0):
            for di in range(4):
                i = i0 + di
                ph_v[i, :] = jnp.maximum(
                    pa_v[i, :] + pb_v[i, :] + ph_v[i, :] + bias, 0.0)

        pltpu.sync_copy(ph_v, table_sh.at[tsl])

        @pl.when(cid == 0)
        def _():
            pltpu.sync_copy(ph_v, h_hbm.at[tsl])

        pltpu.make_async_copy(ei_hbm.at[0, wid], src_v, sems[0]).wait()
        pltpu.make_async_copy(ei_hbm.at[1, wid], dst_v, sems[1]).wait()
        _wait_zero(zero_v, agg_sh, sid, sems, 2)
        plsc.subcore_barrier()
        _seg_sum_loop(table_sh, src_v, dst_v, rows_v, agg_sh, sems)
        plsc.subcore_barrier()
        pltpu.sync_copy(agg_sh.at[pl.ds(sid * RPT, RPT)],
                        out_hbm.at[cid, pl.ds(sid * RPT, RPT)])

    return k(parts1, r1, b1, ei, zeros_chunk)


def _tc_pre(x_w, w_rel_blk, w_root_blk):
    def body(x_ref, a_ref, b_ref, p_ref, r_ref):
        xv = x_ref[...]
        p_ref[...] = jnp.dot(xv, a_ref[...], preferred_element_type=jnp.float32)
        r_ref[...] = jnp.dot(xv, b_ref[...], preferred_element_type=jnp.float32)

    return pl.pallas_call(
        body,
        out_shape=(jax.ShapeDtypeStruct((WROWS, 128), jnp.float32),
                   jax.ShapeDtypeStruct((WROWS, 128), jnp.float32)),
    )(x_w, w_rel_blk, w_root_blk)


def _tc_post(parts_w, h_w, w2_rel_blk, b2_tile, w2_root_blk):
    def body(parts_ref, h_ref, wr_ref, b_ref, wo_ref, o_ref):
        agg_w = parts_ref[0] + parts_ref[1]
        o_ref[...] = (
            jnp.dot(agg_w, wr_ref[...], preferred_element_type=jnp.float32)
            + jnp.dot(h_ref[...], wo_ref[...],
                      preferred_element_type=jnp.float32)
            + b_ref[...])

    return pl.pallas_call(
        body,
        out_shape=jax.ShapeDtypeStruct((WROWS, 2 * NPACK), jnp.float32),
    )(parts_w, h_w, w2_rel_blk, b2_tile, w2_root_blk)


def kernel(x, edge_index, W1_rel, b1, W1_root, W2_rel, b2, W2_root):
    npad = EPAD - EDGES
    ei_pad = jnp.concatenate(
        [jnp.zeros((1, npad), jnp.int32),
         jnp.full((1, npad), NODES, jnp.int32)])
    ei = jnp.concatenate(
        [edge_index.astype(jnp.int32), ei_pad], axis=1).reshape(
            2, NW, NCH, CH)
    zeros_chunk = jnp.zeros((CH, D), jnp.float32)
    eye8 = jnp.eye(NPACK, dtype=jnp.float32)
    w1_rel_blk = jnp.kron(eye8, W1_rel)
    w1_root_blk = jnp.kron(eye8, W1_root)
    w2_rel_blk = jnp.kron(eye8, W2_rel)
    w2_root_blk = jnp.kron(eye8, W2_root)
    b2_tile = jnp.tile(b2, NPACK)
    x_w = jnp.concatenate(
        [x, jnp.zeros((NP - NODES, x.shape[1]), jnp.float32)]).reshape(
            WROWS, NPACK * x.shape[1])

    p1w, r1w = _tc_pre(x_w, w1_rel_blk, w1_root_blk)
    parts1 = _seg_sum_sc(p1w.reshape(NP, D), ei, zeros_chunk)
    parts2, h = _relu_seg_sum_sc(
        parts1, r1w.reshape(NP, D), b1, ei, zeros_chunk)
    out_w = _tc_post(parts2.reshape(NC, WROWS, 128), h.reshape(WROWS, 128),
                     w2_rel_blk, b2_tile, w2_root_blk)
    return out_w.reshape(NP * 2)[:NODES * 2].reshape(NODES, 2)
